# baseline (speedup 1.0000x reference)
import functools

import jax
import jax.numpy as jnp
from jax import lax
from jax.experimental import pallas as pl
from jax.experimental.pallas import tpu as pltpu
from jax.experimental.pallas import tpu_sc as plsc

_INFO = plsc.get_sparse_core_info()
_NC = _INFO.num_cores
_NS = _INFO.num_subcores
_NW = _NC * _NS

_CHUNK = 640


def _gather_flat(idx_flat, emb_table, n_rows, d):
    b_per_w = n_rows // _NW
    nchunks = b_per_w // _CHUNK
    assert nchunks * _CHUNK == b_per_w

    mesh = plsc.VectorSubcoreMesh(core_axis_name="c", subcore_axis_name="s")

    @functools.partial(
        pl.kernel,
        out_type=jax.ShapeDtypeStruct((n_rows, d), jnp.float32),
        mesh=mesh,
        scratch_types=[
            pltpu.VMEM((b_per_w,), jnp.int32),
            pltpu.VMEM((_CHUNK, d), jnp.float32),
            pltpu.VMEM((_CHUNK, d), jnp.float32),
            pltpu.SemaphoreType.DMA,
            pltpu.SemaphoreType.DMA,
            pltpu.SemaphoreType.DMA,
            pltpu.SemaphoreType.DMA,
        ],
        compiler_params=pltpu.CompilerParams(use_tc_tiling_on_sc=False),
    )
    def k(idx_hbm, table_hbm, out_hbm, idx_v, rows0, rows1, g0, g1, o0, o1):
        wid = lax.axis_index("s") * _NC + lax.axis_index("c")
        base = wid * b_per_w
        pltpu.sync_copy(idx_hbm.at[pl.ds(base, b_per_w)], idx_v)

        rows = (rows0, rows1)
        gsem = (g0, g1)
        osem = (o0, o1)

        def start_gather(j):
            b = j % 2
            return pltpu.async_copy(
                table_hbm.at[idx_v.at[pl.ds(j * _CHUNK, _CHUNK)]],
                rows[b], gsem[b])

        gh = [None, None]
        oh = [None, None]
        gh[0] = start_gather(0)
        for j in range(nchunks):
            b = j % 2
            nb = (j + 1) % 2
            if j + 1 < nchunks:
                if oh[nb] is not None:
                    oh[nb].wait()
                gh[nb] = start_gather(j + 1)
            gh[b].wait()
            oh[b] = pltpu.async_copy(
                rows[b], out_hbm.at[pl.ds(base + j * _CHUNK, _CHUNK)],
                osem[b])
        for h in oh:
            if h is not None:
                h.wait()

    return k(idx_flat, emb_table)


def kernel(input_ids, emb_table):
    bsz, seq = input_ids.shape
    vocab, d = emb_table.shape
    n_rows = bsz * seq
    idx_flat = input_ids.reshape(n_rows).astype(jnp.int32)
    out = _gather_flat(idx_flat, emb_table, n_rows, d)
    return out.reshape(bsz, seq, d)


# speedup vs baseline: 1.0061x; 1.0061x over previous
import functools

import jax
import jax.numpy as jnp
from jax import lax
from jax.experimental import pallas as pl
from jax.experimental.pallas import tpu as pltpu
from jax.experimental.pallas import tpu_sc as plsc

_INFO = plsc.get_sparse_core_info()
_NC = _INFO.num_cores
_NS = _INFO.num_subcores
_NW = _NC * _NS

_CHUNK = 320


def _gather_flat(idx_flat, table_pad, n_rows, dpad):
    b_per_w = n_rows // _NW
    nchunks = b_per_w // _CHUNK
    assert nchunks * _CHUNK == b_per_w

    mesh = plsc.VectorSubcoreMesh(core_axis_name="c", subcore_axis_name="s")

    @functools.partial(
        pl.kernel,
        out_type=jax.ShapeDtypeStruct((n_rows, dpad), jnp.float32),
        mesh=mesh,
        scratch_types=[
            pltpu.VMEM((b_per_w,), jnp.int32),
            pltpu.VMEM((_CHUNK, dpad), jnp.float32),
            pltpu.VMEM((_CHUNK, dpad), jnp.float32),
            pltpu.SemaphoreType.DMA,
            pltpu.SemaphoreType.DMA,
            pltpu.SemaphoreType.DMA,
            pltpu.SemaphoreType.DMA,
        ],
        compiler_params=pltpu.CompilerParams(use_tc_tiling_on_sc=False),
    )
    def k(idx_hbm, table_hbm, out_hbm, idx_v, rows0, rows1, g0, g1, o0, o1):
        wid = lax.axis_index("s") * _NC + lax.axis_index("c")
        base = wid * b_per_w
        pltpu.sync_copy(idx_hbm.at[pl.ds(base, b_per_w)], idx_v)

        rows = (rows0, rows1)
        gsem = (g0, g1)
        osem = (o0, o1)

        def start_gather(j):
            b = j % 2
            return pltpu.async_copy(
                table_hbm.at[idx_v.at[pl.ds(j * _CHUNK, _CHUNK)]],
                rows[b], gsem[b])

        gh = [None, None]
        oh = [None, None]
        gh[0] = start_gather(0)
        for j in range(nchunks):
            b = j % 2
            nb = (j + 1) % 2
            if j + 1 < nchunks:
                if oh[nb] is not None:
                    oh[nb].wait()
                gh[nb] = start_gather(j + 1)
            gh[b].wait()
            oh[b] = pltpu.async_copy(
                rows[b], out_hbm.at[pl.ds(base + j * _CHUNK, _CHUNK)],
                osem[b])
        for h in oh:
            if h is not None:
                h.wait()

    return k(idx_flat, table_pad)


def kernel(input_ids, emb_table):
    bsz, seq = input_ids.shape
    vocab, d = emb_table.shape
    dpad = 128
    n_rows = bsz * seq
    idx_flat = input_ids.reshape(n_rows).astype(jnp.int32)
    table_pad = jnp.pad(emb_table, ((0, 0), (0, dpad - d)))
    out = _gather_flat(idx_flat, table_pad, n_rows, dpad)
    return out[:, :d].reshape(bsz, seq, d)
